# baseline (speedup 1.0000x reference)
import functools

import jax
import jax.numpy as jnp
from jax import lax
from jax.experimental import pallas as pl
from jax.experimental.pallas import tpu as pltpu
from jax.experimental.pallas import tpu_sc as plsc

F = 26
V = 100000
E = 32
OUT = 128
B = 16384
EPS = 1e-5

NW = 32
TCOL = 7
FPAD = TCOL * 4
SLOTS = B * FPAD
IDX_MINOR = 128
TILE_ROWS = 8
CHUNK = TILE_ROWS * IDX_MINOR
PER_W_TILES = SLOTS // IDX_MINOR // NW
STEPS = PER_W_TILES // TILE_ROWS


def _sc_gather(table_flat, idx2):
    mesh = plsc.VectorSubcoreMesh(core_axis_name="c", subcore_axis_name="s")

    @functools.partial(
        pl.kernel,
        mesh=mesh,
        out_type=jax.ShapeDtypeStruct((SLOTS, E), jnp.float32),
        scratch_types=[
            pltpu.VMEM((TILE_ROWS, IDX_MINOR), jnp.int32),
            pltpu.VMEM((CHUNK, E), jnp.float32),
            pltpu.SemaphoreType.DMA,
        ],
        compiler_params=pltpu.CompilerParams(use_tc_tiling_on_sc=False),
    )
    def k(tbl, idx_hbm, out_hbm, idx_v, rows_v, sem):
        wid = lax.axis_index("s") * 2 + lax.axis_index("c")
        tile_base = wid * PER_W_TILES

        def body(i, carry):
            t0 = tile_base + i * TILE_ROWS
            pltpu.sync_copy(idx_hbm.at[pl.ds(t0, TILE_ROWS)], idx_v)
            cps = [
                pltpu.async_copy(
                    tbl.at[idx_v.at[j]],
                    rows_v.at[pl.ds(j * IDX_MINOR, IDX_MINOR)],
                    sem,
                )
                for j in range(TILE_ROWS)
            ]
            for cp in cps:
                cp.wait()
            pltpu.sync_copy(rows_v, out_hbm.at[pl.ds(t0 * IDX_MINOR, CHUNK)])
            return carry

        lax.fori_loop(0, STEPS, body, 0)

    return k(table_flat, idx2)


def _tc_proj(emb4, W4, b, gamma, beta):
    TRB = 64

    def body(e_ref, w_ref, b_ref, g_ref, bt_ref, o_ref):
        acc = jnp.zeros((TRB * 8, OUT), jnp.float32)
        for tc in range(TCOL):
            blk = e_ref[:, tc].reshape(TRB * 8, 128)
            acc += jnp.dot(blk, w_ref[tc], preferred_element_type=jnp.float32)
        h = jnp.maximum(acc + b_ref[...], 0.0)
        m = jnp.mean(h, axis=-1, keepdims=True)
        c = h - m
        v = jnp.mean(c * c, axis=-1, keepdims=True)
        o_ref[...] = c * lax.rsqrt(v + EPS) * g_ref[...] + bt_ref[...]

    return pl.pallas_call(
        body,
        grid=(B // (TRB * 8),),
        in_specs=[
            pl.BlockSpec((TRB, TCOL, 8, 128), lambda i: (i, 0, 0, 0)),
            pl.BlockSpec((TCOL, 128, OUT), lambda i: (0, 0, 0)),
            pl.BlockSpec((1, OUT), lambda i: (0, 0)),
            pl.BlockSpec((1, OUT), lambda i: (0, 0)),
            pl.BlockSpec((1, OUT), lambda i: (0, 0)),
        ],
        out_specs=pl.BlockSpec((TRB * 8, OUT), lambda i: (i, 0)),
        out_shape=jax.ShapeDtypeStruct((B, OUT), jnp.float32),
    )(emb4, W4, b.reshape(1, OUT), gamma.reshape(1, OUT), beta.reshape(1, OUT))


def kernel(x, tables, W, b, gamma, beta):
    fr = jnp.arange(FPAD, dtype=jnp.int32)
    offs = jnp.where(fr < F, fr * V, 0)[None, :]
    xpad = jnp.pad(x.astype(jnp.int32), ((0, 0), (0, FPAD - F)))
    idx = xpad + offs
    idx = idx.reshape(B // 8, 8, TCOL, 4).transpose(0, 2, 1, 3)
    idx2 = idx.reshape(SLOTS // IDX_MINOR, IDX_MINOR)

    emb = _sc_gather(tables.reshape(F * V, E), idx2)
    emb4 = emb.reshape(B // 8, TCOL, 8, 128)

    Wpad = jnp.pad(W, ((0, TCOL * 128 - F * E), (0, 0)))
    W4 = Wpad.reshape(TCOL, 128, OUT)
    return _tc_proj(emb4, W4, b, gamma, beta)
